# Initial kernel scaffold; baseline (speedup 1.0000x reference)
import jax
import jax.numpy as jnp
from jax.experimental import pallas as pl


def kernel(feats, logit):
    raise NotImplementedError("write your pallas kernel here")


# speedup vs baseline: 1.4436x; 1.4436x over previous
import functools

import jax
import jax.numpy as jnp
from jax import lax
from jax.experimental import pallas as pl
from jax.experimental.pallas import tpu as pltpu
from jax.experimental.pallas import tpu_sc as plsc

B = 4
S = 8192
D = 768
K = 2048
LOG2S = 13


def _sort_body(key_ref, a0_ref, a1_ref, gidx_ref, l0s_ref, l1s_ref):
    key = key_ref[...]
    a0 = a0_ref[...]
    a1 = a1_ref[...]
    it = lax.broadcasted_iota(jnp.int32, (B, S), 1)
    idx = it

    for klog in range(1, LOG2S + 1):
        kk = 1 << klog
        for jlog in range(klog - 1, -1, -1):
            j = 1 << jlog
            is_hi = (it & j) != 0
            dir_up = (it & kk) == 0

            def partner(x, j=j, is_hi=is_hi):
                return jnp.where(is_hi, jnp.roll(x, j, axis=1),
                                 jnp.roll(x, -j, axis=1))

            pk = partner(key)
            pi = partner(idx)
            pa0 = partner(a0)
            pa1 = partner(a1)
            less = (key > pk) | ((key == pk) & (idx < pi))
            keep = jnp.logical_xor(less, is_hi) == dir_up
            key = jnp.where(keep, key, pk)
            idx = jnp.where(keep, idx, pi)
            a0 = jnp.where(keep, a0, pa0)
            a1 = jnp.where(keep, a1, pa1)

    row = lax.broadcasted_iota(jnp.int32, (B, K), 0)
    gidx_ref[...] = idx[:, :K] + row * S
    l0s_ref[...] = a0
    l1s_ref[...] = a1


_sort_call = pl.pallas_call(
    _sort_body,
    out_shape=(
        jax.ShapeDtypeStruct((B, K), jnp.int32),
        jax.ShapeDtypeStruct((B, S), jnp.float32),
        jax.ShapeDtypeStruct((B, S), jnp.float32),
    ),
)


_NC, _NS = 2, 16
_NW = _NC * _NS
_RPW = (B * K) // _NW
_CHUNK = 64
_NCH = _RPW // _CHUNK

@functools.cache
def _make_sc_gather():
    mesh = plsc.VectorSubcoreMesh(core_axis_name="c", subcore_axis_name="s")

    @functools.partial(
        pl.kernel,
        mesh=mesh,
        out_type=jax.ShapeDtypeStruct((B * K, D), jnp.float32),
        scratch_types=[
            pltpu.VMEM((_RPW,), jnp.int32),
            pltpu.VMEM((_CHUNK, D), jnp.float32),
            pltpu.VMEM((_CHUNK, D), jnp.float32),
            pltpu.SemaphoreType.DMA,
            pltpu.SemaphoreType.DMA,
        ],
    )
    def sc_gather(table_hbm, idx_hbm, out_hbm, idx_v, buf0, buf1, sem0, sem1):
        wid = lax.axis_index("s") * _NC + lax.axis_index("c")
        base = wid * _RPW
        pltpu.sync_copy(idx_hbm.at[pl.ds(base, _RPW)], idx_v)
        bufs = (buf0, buf1)
        sems = (sem0, sem1)
        cps = [None] * _NCH
        cps[0] = pltpu.async_copy(
            table_hbm.at[idx_v.at[pl.ds(0, _CHUNK)]], buf0, sem0)
        for c in range(_NCH):
            if c + 1 < _NCH:
                cps[c + 1] = pltpu.async_copy(
                    table_hbm.at[idx_v.at[pl.ds((c + 1) * _CHUNK, _CHUNK)]],
                    bufs[(c + 1) % 2], sems[(c + 1) % 2])
            cps[c].wait()
            pltpu.sync_copy(bufs[c % 2],
                            out_hbm.at[pl.ds(base + c * _CHUNK, _CHUNK)])

    return sc_gather


def kernel(feats, logit):
    probs = jax.nn.softmax(logit, axis=-1)
    maxp = jnp.max(probs, axis=-1)
    l0 = logit[..., 0]
    l1 = logit[..., 1]
    gidx, l0s, l1s = _sort_call(maxp, l0, l1)
    sf = _make_sc_gather()(feats.reshape(B * S, D), gidx.reshape(B * K))
    sf = sf.reshape(B, K, D)
    preds_1 = jnp.stack([l0s[:, :K], l1s[:, :K]], axis=-1)
    preds_0 = jnp.stack([l0s[:, K:], l1s[:, K:]], axis=-1)
    return sf, preds_1, preds_0


# speedup vs baseline: 1.7588x; 1.2183x over previous
import functools

import jax
import jax.numpy as jnp
from jax import lax
from jax.experimental import pallas as pl
from jax.experimental.pallas import tpu as pltpu
from jax.experimental.pallas import tpu_sc as plsc

B = 4
S = 8192
D = 768
K = 2048
LOG2S = 13


R = 8
C = S // R


def _sort_body(key_ref, a0_ref, a1_ref, gidx_ref, l0s_ref, l1s_ref):
    key = key_ref[...]
    a0 = a0_ref[...]
    a1 = a1_ref[...]
    g = lax.broadcasted_iota(jnp.int32, (B * R, C), 0)
    cc = lax.broadcasted_iota(jnp.int32, (B * R, C), 1)
    it = (g & (R - 1)) * C + cc
    idx = it

    for klog in range(1, LOG2S + 1):
        kk = 1 << klog
        for jlog in range(klog - 1, -1, -1):
            j = 1 << jlog
            is_hi = (it & j) != 0
            dir_up = (it & kk) == 0

            def partner(x, j=j, is_hi=is_hi):
                if j < C:
                    return jnp.where(is_hi, jnp.roll(x, j, axis=1),
                                     jnp.roll(x, -j, axis=1))
                d = j // C
                return jnp.where(is_hi, jnp.roll(x, d, axis=0),
                                 jnp.roll(x, -d, axis=0))

            pk = partner(key)
            pi = partner(idx)
            pa0 = partner(a0)
            pa1 = partner(a1)
            less = (key > pk) | ((key == pk) & (idx < pi))
            keep = jnp.logical_xor(less, is_hi) == dir_up
            key = jnp.where(keep, key, pk)
            idx = jnp.where(keep, idx, pi)
            a0 = jnp.where(keep, a0, pa0)
            a1 = jnp.where(keep, a1, pa1)

    gidx_ref[...] = idx + (g >> 3) * S
    l0s_ref[...] = a0
    l1s_ref[...] = a1


_sort_call = pl.pallas_call(
    _sort_body,
    out_shape=(
        jax.ShapeDtypeStruct((B * R, C), jnp.int32),
        jax.ShapeDtypeStruct((B * R, C), jnp.float32),
        jax.ShapeDtypeStruct((B * R, C), jnp.float32),
    ),
)


_NC, _NS = 2, 16
_NW = _NC * _NS
_RPW = (B * K) // _NW
_CHUNK = 64
_NCH = _RPW // _CHUNK

@functools.cache
def _make_sc_gather():
    mesh = plsc.VectorSubcoreMesh(core_axis_name="c", subcore_axis_name="s")

    @functools.partial(
        pl.kernel,
        mesh=mesh,
        out_type=jax.ShapeDtypeStruct((B * K, D), jnp.float32),
        scratch_types=[
            pltpu.VMEM((_RPW,), jnp.int32),
            pltpu.VMEM((_CHUNK, D), jnp.float32),
            pltpu.VMEM((_CHUNK, D), jnp.float32),
            pltpu.SemaphoreType.DMA,
            pltpu.SemaphoreType.DMA,
        ],
    )
    def sc_gather(table_hbm, idx_hbm, out_hbm, idx_v, buf0, buf1, sem0, sem1):
        wid = lax.axis_index("s") * _NC + lax.axis_index("c")
        base = wid * _RPW
        pltpu.sync_copy(idx_hbm.at[pl.ds(base, _RPW)], idx_v)
        bufs = (buf0, buf1)
        sems = (sem0, sem1)
        cps = [None] * _NCH
        cps[0] = pltpu.async_copy(
            table_hbm.at[idx_v.at[pl.ds(0, _CHUNK)]], buf0, sem0)
        for c in range(_NCH):
            if c + 1 < _NCH:
                cps[c + 1] = pltpu.async_copy(
                    table_hbm.at[idx_v.at[pl.ds((c + 1) * _CHUNK, _CHUNK)]],
                    bufs[(c + 1) % 2], sems[(c + 1) % 2])
            cps[c].wait()
            pltpu.sync_copy(bufs[c % 2],
                            out_hbm.at[pl.ds(base + c * _CHUNK, _CHUNK)])

    return sc_gather


def kernel(feats, logit):
    probs = jax.nn.softmax(logit, axis=-1)
    maxp = jnp.max(probs, axis=-1)
    l0 = logit[..., 0]
    l1 = logit[..., 1]
    gidx2, l0s2, l1s2 = _sort_call(
        maxp.reshape(B * R, C), l0.reshape(B * R, C), l1.reshape(B * R, C))
    gidx = gidx2.reshape(B, S)[:, :K]
    l0s = l0s2.reshape(B, S)
    l1s = l1s2.reshape(B, S)
    sf = _make_sc_gather()(feats.reshape(B * S, D), gidx.reshape(B * K))
    sf = sf.reshape(B, K, D)
    preds_1 = jnp.stack([l0s[:, :K], l1s[:, :K]], axis=-1)
    preds_0 = jnp.stack([l0s[:, K:], l1s[:, K:]], axis=-1)
    return sf, preds_1, preds_0
